# baseline (speedup 1.0000x reference)
import jax
import jax.numpy as jnp
from jax.experimental import pallas as pl
from jax.experimental.pallas import tpu as pltpu


def kernel(inputs, pos_table):
    N, T = inputs.shape
    H = pos_table.shape[1]
    scale = float(H) ** 0.5

    BN = 256
    NB = N // BN

    def body(tab_ref, out_ref):
        out_ref[...] = jnp.broadcast_to(tab_ref[...] * scale, out_ref.shape)

    out = pl.pallas_call(
        body,
        grid=(NB,),
        in_specs=[pl.BlockSpec((T, H), lambda i: (0, 0))],
        out_specs=pl.BlockSpec((BN, T, H), lambda i: (i, 0, 0)),
        out_shape=jax.ShapeDtypeStruct((N, T, H), jnp.float32),
    )(pos_table)
    return out
